# Initial kernel scaffold; baseline (speedup 1.0000x reference)
import jax
import jax.numpy as jnp
from jax.experimental import pallas as pl


def kernel(x, edge_index, W_l0, b_l0, W_r0, W_l1, b_l1, W_r1, W_out, b_out):
    raise NotImplementedError("write your pallas kernel here")


# speedup vs baseline: 3.7995x; 3.7995x over previous
import jax
import jax.numpy as jnp
from jax import lax
from jax.experimental import pallas as pl
from jax.experimental.pallas import tpu as pltpu
from jax.experimental.pallas import tpu_sc as plsc

N = 10000
E = 320000
D = 128

NC = 2
NS = 16
NW = NC * NS
K = 128
CH = -(-E // (NW * K))
EP = NW * CH * K
NP = 10240
RPT = NP // NS
RB = 1024

_f32 = jnp.float32


def _sc_agg_phase(y_hbm, src_hbm, dst_hbm, z2d_hbm, out_hbm,
                  sidx, didx, rows, aggs, sem, c, s, wid):
    for b in range(RPT // K):
        sl = pl.ds(s * RPT + b * K, K)
        pltpu.sync_copy(z2d_hbm, aggs.at[sl])
    plsc.subcore_barrier()

    def chunk(j, carry):
        base = pl.multiple_of((wid * CH + j) * K, K)
        pltpu.sync_copy(src_hbm.at[pl.ds(base, K)], sidx)
        pltpu.sync_copy(dst_hbm.at[pl.ds(base, K)], didx)
        pltpu.async_copy(y_hbm.at[sidx], rows, sem).wait()
        pltpu.sync_copy(rows, aggs.at[didx], add=True)
        return carry

    lax.fori_loop(0, CH, chunk, 0)
    plsc.subcore_barrier()

    for b in range(RPT // K):
        sl = pl.ds(s * RPT + b * K, K)
        pltpu.sync_copy(aggs.at[sl], out_hbm.at[c, sl])


def _sc_deg_body(y_hbm, src_hbm, dst_hbm, z2d_hbm, ones_hbm,
                 agg_out, deg_out,
                 sidx, didx, rows, ones, aggs, sem):
    c = lax.axis_index("c")
    s = lax.axis_index("s")
    wid = s * NC + c

    pltpu.sync_copy(ones_hbm, ones)
    for b in range(RPT // K):
        sl = pl.ds(s * RPT + b * K, K)
        pltpu.sync_copy(z2d_hbm, aggs.at[sl])
    plsc.subcore_barrier()

    def dchunk(j, carry):
        base = pl.multiple_of((wid * CH + j) * K, K)
        pltpu.sync_copy(dst_hbm.at[pl.ds(base, K)], didx)
        pltpu.sync_copy(ones, aggs.at[didx], add=True)
        return carry

    lax.fori_loop(0, CH, dchunk, 0)
    plsc.subcore_barrier()

    for b in range(RPT // K):
        sl = pl.ds(s * RPT + b * K, K)
        pltpu.sync_copy(aggs.at[sl], deg_out.at[c, sl])
    plsc.subcore_barrier()

    _sc_agg_phase(y_hbm, src_hbm, dst_hbm, z2d_hbm, agg_out,
                  sidx, didx, rows, aggs, sem, c, s, wid)


def _sc_plain_body(y_hbm, src_hbm, dst_hbm, z2d_hbm, agg_out,
                   sidx, didx, rows, aggs, sem):
    c = lax.axis_index("c")
    s = lax.axis_index("s")
    wid = s * NC + c
    _sc_agg_phase(y_hbm, src_hbm, dst_hbm, z2d_hbm, agg_out,
                  sidx, didx, rows, aggs, sem, c, s, wid)


_sc_mesh = plsc.VectorSubcoreMesh(
    core_axis_name="c", subcore_axis_name="s", num_cores=NC, num_subcores=NS)

_sc_scatter_deg = pl.kernel(
    _sc_deg_body,
    out_type=(
        jax.ShapeDtypeStruct((NC, NP, D), _f32),
        jax.ShapeDtypeStruct((NC, NP, D), _f32),
    ),
    mesh=_sc_mesh,
    scratch_types=(
        pltpu.VMEM((K,), jnp.int32),
        pltpu.VMEM((K,), jnp.int32),
        pltpu.VMEM((K, D), _f32),
        pltpu.VMEM((K, D), _f32),
        pltpu.VMEM_SHARED((NP, D), _f32),
        pltpu.SemaphoreType.DMA,
    ),
)

_sc_scatter = pl.kernel(
    _sc_plain_body,
    out_type=jax.ShapeDtypeStruct((NC, NP, D), _f32),
    mesh=_sc_mesh,
    scratch_types=(
        pltpu.VMEM((K,), jnp.int32),
        pltpu.VMEM((K,), jnp.int32),
        pltpu.VMEM((K, D), _f32),
        pltpu.VMEM_SHARED((NP, D), _f32),
        pltpu.SemaphoreType.DMA,
    ),
)


def _tc_pre_body(x_ref, wl_ref, wr_ref, b_ref, y_ref, z_ref):
    xb = x_ref[...]
    y_ref[...] = jnp.dot(xb, wl_ref[...], preferred_element_type=_f32,
                         precision=lax.Precision.HIGHEST)
    z_ref[...] = jnp.dot(xb, wr_ref[...], preferred_element_type=_f32,
                         precision=lax.Precision.HIGHEST) + b_ref[...]


_tc_pre = pl.pallas_call(
    _tc_pre_body,
    grid=(NP // RB,),
    in_specs=[
        pl.BlockSpec((RB, D), lambda i: (i, 0)),
        pl.BlockSpec((D, D), lambda i: (0, 0)),
        pl.BlockSpec((D, D), lambda i: (0, 0)),
        pl.BlockSpec((1, D), lambda i: (0, 0)),
    ],
    out_specs=[
        pl.BlockSpec((RB, D), lambda i: (i, 0)),
        pl.BlockSpec((RB, D), lambda i: (i, 0)),
    ],
    out_shape=[
        jax.ShapeDtypeStruct((NP, D), _f32),
        jax.ShapeDtypeStruct((NP, D), _f32),
    ],
)


def _mean_relu(agg_ref, deg_ref, z_ref):
    agg = agg_ref[0] + agg_ref[1]
    deg = jnp.max(deg_ref[0] + deg_ref[1], axis=1, keepdims=True)
    h = agg / jnp.maximum(deg, 1.0) + z_ref[...]
    return jnp.maximum(h, 0.0)


def _tc_mid_body(agg_ref, deg_ref, z_ref, wl_ref, wr_ref, b_ref,
                 y_ref, z1_ref):
    h = _mean_relu(agg_ref, deg_ref, z_ref)
    y_ref[...] = jnp.dot(h, wl_ref[...], preferred_element_type=_f32,
                         precision=lax.Precision.HIGHEST)
    z1_ref[...] = jnp.dot(h, wr_ref[...], preferred_element_type=_f32,
                          precision=lax.Precision.HIGHEST) + b_ref[...]


_tc_mid = pl.pallas_call(
    _tc_mid_body,
    grid=(NP // RB,),
    in_specs=[
        pl.BlockSpec((NC, RB, D), lambda i: (0, i, 0)),
        pl.BlockSpec((NC, RB, D), lambda i: (0, i, 0)),
        pl.BlockSpec((RB, D), lambda i: (i, 0)),
        pl.BlockSpec((D, D), lambda i: (0, 0)),
        pl.BlockSpec((D, D), lambda i: (0, 0)),
        pl.BlockSpec((1, D), lambda i: (0, 0)),
    ],
    out_specs=[
        pl.BlockSpec((RB, D), lambda i: (i, 0)),
        pl.BlockSpec((RB, D), lambda i: (i, 0)),
    ],
    out_shape=[
        jax.ShapeDtypeStruct((NP, D), _f32),
        jax.ShapeDtypeStruct((NP, D), _f32),
    ],
)


def _tc_out_body(agg_ref, deg_ref, z_ref, wo_ref, b_ref, o_ref):
    h = _mean_relu(agg_ref, deg_ref, z_ref)
    o_ref[...] = jnp.dot(h, wo_ref[...], preferred_element_type=_f32,
                         precision=lax.Precision.HIGHEST) + b_ref[...]


_tc_out = pl.pallas_call(
    _tc_out_body,
    grid=(NP // RB,),
    in_specs=[
        pl.BlockSpec((NC, RB, D), lambda i: (0, i, 0)),
        pl.BlockSpec((NC, RB, D), lambda i: (0, i, 0)),
        pl.BlockSpec((RB, D), lambda i: (i, 0)),
        pl.BlockSpec((D, D), lambda i: (0, 0)),
        pl.BlockSpec((1, D), lambda i: (0, 0)),
    ],
    out_specs=pl.BlockSpec((RB, D), lambda i: (i, 0)),
    out_shape=jax.ShapeDtypeStruct((NP, D), _f32),
)


def kernel(x, edge_index, W_l0, b_l0, W_r0, W_l1, b_l1, W_r1, W_out, b_out):
    ei = edge_index.astype(jnp.int32)
    src = jnp.concatenate([ei[0], jnp.zeros((EP - E,), jnp.int32)])
    dst = jnp.concatenate([ei[1], jnp.full((EP - E,), N, jnp.int32)])
    xp = jnp.pad(x, ((0, NP - N), (0, 0)))
    z2d = jnp.zeros((K, D), _f32)
    ones2d = jnp.ones((K, D), _f32)

    y0, z0 = _tc_pre(xp, W_l0, W_r0, b_l0.reshape(1, D))
    agg0, deg = _sc_scatter_deg(y0, src, dst, z2d, ones2d)
    y1, z1 = _tc_mid(agg0, deg, z0, W_l1, W_r1, b_l1.reshape(1, D))
    agg1 = _sc_scatter(y1, src, dst, z2d)
    out = _tc_out(agg1, deg, z1, W_out, b_out.reshape(1, D))
    return out[:N]
